# baseline (speedup 1.0000x reference)
import functools

import jax
import jax.numpy as jnp
from jax import lax
from jax.experimental import pallas as pl
from jax.experimental.pallas import tpu as pltpu
from jax.experimental.pallas import tpu_sc as plsc

_VOCAB = 1000000
_C = 200
_D = 64
_B = 1024
_FLAT = _B * _C
_CHUNK = 64
_NC, _NS = 2, 16
_NW = _NC * _NS
_CPW = _FLAT // (_CHUNK * _NW)
_IPW = _CPW * _CHUNK
_NBUF = 5
_BLKS = _CPW // _NBUF
_LANES = 16
_FD = 2 * _D


def _sc_gather_add(text_flat, table2, pos):
    mesh = plsc.VectorSubcoreMesh(core_axis_name="c", subcore_axis_name="s",
                                  num_cores=_NC, num_subcores=_NS)

    @functools.partial(
        pl.kernel,
        out_type=jax.ShapeDtypeStruct((_FLAT, _D), jnp.float32),
        mesh=mesh,
        scratch_types=[
            pltpu.VMEM((_IPW,), jnp.int32),
            pltpu.VMEM((_IPW,), jnp.int32),
            pltpu.VMEM((_C, _D), jnp.float32),
            pltpu.VMEM((_NBUF, _CHUNK, _FD), jnp.float32),
            pltpu.VMEM((_NBUF, _CHUNK, _D), jnp.float32),
            pltpu.SemaphoreType.DMA((_NBUF,)),
            pltpu.SemaphoreType.DMA((_NBUF,)),
        ],
        compiler_params=pltpu.CompilerParams(use_tc_tiling_on_sc=True,
                                             needs_layout_passes=False),
    )
    def k(text_hbm, table_hbm, pos_hbm, out_hbm,
          raw_v, idx_v, pos_v, fbuf_v, obuf_v, gsem, ssem):
        wid = lax.axis_index("s") * _NC + lax.axis_index("c")
        pltpu.sync_copy(pos_hbm, pos_v)
        base = wid * _IPW
        pltpu.sync_copy(text_hbm.at[pl.ds(base, _IPW)], raw_v)

        def halve(m, _):
            sl = pl.ds(m * _LANES, _LANES)
            idx_v[sl] = lax.shift_right_logical(raw_v[sl], 1)
            return ()

        lax.fori_loop(0, _IPW // _LANES, halve, (), unroll=8)
        chunk0 = wid * _CPW

        def gather_start(j, b):
            ioff = pl.multiple_of(j * _CHUNK, _CHUNK)
            pltpu.async_copy(table_hbm.at[idx_v.at[pl.ds(ioff, _CHUNK)]],
                             fbuf_v.at[b], gsem.at[b])

        def gather_wait(b):
            pltpu.make_async_copy(table_hbm.at[idx_v.at[pl.ds(0, _CHUNK)]],
                                  fbuf_v.at[b], gsem.at[b]).wait()

        def store_start(j, b):
            off = pl.multiple_of((chunk0 + j) * _CHUNK, _CHUNK)
            pltpu.async_copy(obuf_v.at[b], out_hbm.at[pl.ds(off, _CHUNK)],
                             ssem.at[b])

        def store_wait(b):
            pltpu.make_async_copy(obuf_v.at[b], out_hbm.at[pl.ds(0, _CHUNK)],
                                  ssem.at[b]).wait()

        def step(j, b, first_block, last_block):
            gather_wait(b)
            if not first_block:
                store_wait(b)
            ioff = pl.multiple_of(j * _CHUNK, _CHUNK)
            p0 = lax.rem(j * _CHUNK, _C)

            def row_body(r, p):
                rsplat = jnp.full((_LANES,), ioff + r, jnp.int32)
                msk = (plsc.load_gather(raw_v, [rsplat]) & 1) == 1
                for kk in range(_D // _LANES):
                    sl = pl.ds(kk * _LANES, _LANES)
                    lo = fbuf_v[b, r, sl]
                    hi = fbuf_v[b, r, pl.ds(_D + kk * _LANES, _LANES)]
                    obuf_v[b, r, sl] = jnp.where(msk, hi, lo) + pos_v[p, sl]
                p = p + 1
                return lax.select(p == _C, 0, p)

            lax.fori_loop(0, _CHUNK, row_body, p0, unroll=2)
            if not last_block:
                gather_start(j + _NBUF, b)
            store_start(j, b)

        for b in range(_NBUF):
            gather_start(b, b)
        for b in range(_NBUF):
            step(b, b, True, False)

        def mid_block(jo, _):
            for b in range(_NBUF):
                step(jo * _NBUF + b, b, False, False)
            return ()

        lax.fori_loop(1, _BLKS - 1, mid_block, ())
        for b in range(_NBUF):
            step((_BLKS - 1) * _NBUF + b, b, False, True)
        for b in range(_NBUF):
            store_wait(b)

    return k(text_flat, table2, pos)


_TB = 2
_TPW = 244
_TBLK = _TPW // _TB
_TAILT = 7812


def _sc_build_table(tt, tail):
    mesh = plsc.VectorSubcoreMesh(core_axis_name="c", subcore_axis_name="s",
                                  num_cores=_NC, num_subcores=_NS)

    @functools.partial(
        pl.kernel,
        out_type=jax.ShapeDtypeStruct((_VOCAB // 2, _FD), jnp.float32),
        mesh=mesh,
        scratch_types=[
            pltpu.VMEM((2, _D, _TB * 128), jnp.float32),
            pltpu.VMEM((2, _TB * 64, _FD), jnp.float32),
            pltpu.VMEM((_D, _D), jnp.float32),
            pltpu.SemaphoreType.DMA((2,)),
            pltpu.SemaphoreType.DMA((2,)),
        ],
        compiler_params=pltpu.CompilerParams(use_tc_tiling_on_sc=True,
                                             needs_layout_passes=False),
    )
    def k(tt_hbm, tail_hbm, out_hbm, inb_v, ob_v, tail_v, gsem, ssem):
        wid = lax.axis_index("s") * _NC + lax.axis_index("c")
        t0 = wid * _TPW
        lanes = lax.broadcasted_iota(jnp.int32, (_LANES,), 0)

        def in_start(j, b):
            goff = pl.multiple_of((t0 + j * _TB) * 128, 128)
            pltpu.async_copy(tt_hbm.at[:, pl.ds(goff, _TB * 128)],
                             inb_v.at[b], gsem.at[b])

        def in_wait(b):
            pltpu.make_async_copy(tt_hbm.at[:, pl.ds(0, _TB * 128)],
                                  inb_v.at[b], gsem.at[b]).wait()

        def out_start(j, b):
            off = pl.multiple_of((t0 + j * _TB) * 64, 64)
            pltpu.async_copy(ob_v.at[b], out_hbm.at[pl.ds(off, _TB * 64)],
                             ssem.at[b])

        def out_wait(b):
            pltpu.make_async_copy(ob_v.at[b], out_hbm.at[pl.ds(0, _TB * 64)],
                                  ssem.at[b]).wait()

        def transpose_batch(b):
            def row_body(kf, _):
                for h in range(2):
                    tok = jnp.full((_LANES,), 2 * kf + h, jnp.int32)
                    for kk in range(_D // _LANES):
                        dv = lanes + (kk * _LANES)
                        ob_v[b, kf, pl.ds(h * _D + kk * _LANES, _LANES)] = (
                            plsc.load_gather(inb_v.at[b], [dv, tok]))
                return ()

            lax.fori_loop(0, _TB * 64, row_body, (), unroll=2)

        def step(j, b, first_block, last_block):
            in_wait(b)
            if not first_block:
                out_wait(b)
            transpose_batch(b)
            if not last_block:
                in_start(j + 2, b)
            out_start(j, b)

        for b in range(2):
            in_start(b, b)
        for b in range(2):
            step(b, b, True, False)

        def mid_block(jo, _):
            for b in range(2):
                step(jo * 2 + b, b, False, False)
            return ()

        lax.fori_loop(1, _TBLK - 1, mid_block, ())
        for b in range(2):
            step((_TBLK - 1) * 2 + b, b, False, True)
        for b in range(2):
            out_wait(b)

        @pl.when(wid == _NW - 1)
        def _():
            for e in range(2):
                goff = (_NW * _TPW + e * _TB) * 128
                pltpu.sync_copy(tt_hbm.at[:, pl.ds(goff, _TB * 128)],
                                inb_v.at[0])
                transpose_batch(0)
                pltpu.sync_copy(ob_v.at[0],
                                out_hbm.at[pl.ds((_NW * _TPW + e * _TB) * 64,
                                                 _TB * 64)])
            pltpu.sync_copy(tail_hbm, tail_v)

            def tail_row(kf, _):
                for h in range(2):
                    for kk in range(_D // _LANES):
                        sl = pl.ds(kk * _LANES, _LANES)
                        ob_v[0, kf, pl.ds(h * _D + kk * _LANES, _LANES)] = (
                            tail_v[2 * kf + h, sl])
                return ()

            lax.fori_loop(0, 32, tail_row, ())
            pltpu.sync_copy(ob_v.at[0, pl.ds(0, 32)],
                            out_hbm.at[pl.ds(_TAILT * 64, 32)])

    return k(tt, tail)


def _mask_body(o_ref):
    i = lax.broadcasted_iota(jnp.int32, (_C, _C), 0)
    j = lax.broadcasted_iota(jnp.int32, (_C, _C), 1)
    o_ref[...] = jnp.where(j > i, -jnp.inf, 0.0).astype(jnp.float32)


def _causal_mask():
    return pl.pallas_call(
        _mask_body,
        out_shape=jax.ShapeDtypeStruct((_C, _C), jnp.float32),
    )()


def kernel(text, token_embedding, positional_embedding):
    text_flat = text.astype(jnp.int32).reshape(_FLAT)
    te = token_embedding.astype(jnp.float32)
    tail = lax.slice(te, (_TAILT * 128, 0), (_VOCAB, _D))
    table2 = _sc_build_table(te.T, tail)
    x = _sc_gather_add(text_flat, table2,
                       positional_embedding.astype(jnp.float32))
    return (x.reshape(_B, _C, _D), _causal_mask())


# speedup vs baseline: 1.7832x; 1.7832x over previous
import functools

import jax
import jax.numpy as jnp
from jax import lax
from jax.experimental import pallas as pl
from jax.experimental.pallas import tpu as pltpu
from jax.experimental.pallas import tpu_sc as plsc

_VOCAB = 1000000
_C = 200
_D = 64
_B = 1024
_FLAT = _B * _C
_CHUNK = 64
_NC, _NS = 2, 16
_NW = _NC * _NS
_CPW = _FLAT // (_CHUNK * _NW)
_IPW = _CPW * _CHUNK
_NBUF = 5
_BLKS = _CPW // _NBUF
_LANES = 16
_FD = 2 * _D


def _sc_gather_add(text_flat, table2, pos):
    mesh = plsc.VectorSubcoreMesh(core_axis_name="c", subcore_axis_name="s",
                                  num_cores=_NC, num_subcores=_NS)

    @functools.partial(
        pl.kernel,
        out_type=jax.ShapeDtypeStruct((_FLAT, _D), jnp.float32),
        mesh=mesh,
        scratch_types=[
            pltpu.VMEM((_IPW,), jnp.int32),
            pltpu.VMEM((_IPW,), jnp.int32),
            pltpu.VMEM((_C, _D), jnp.float32),
            pltpu.VMEM((_NBUF, _CHUNK, _FD), jnp.float32),
            pltpu.VMEM((_NBUF, _CHUNK, _D), jnp.float32),
            pltpu.SemaphoreType.DMA((_NBUF,)),
            pltpu.SemaphoreType.DMA((_NBUF,)),
        ],
        compiler_params=pltpu.CompilerParams(use_tc_tiling_on_sc=True,
                                             needs_layout_passes=False),
    )
    def k(text_hbm, table_hbm, pos_hbm, out_hbm,
          raw_v, idx_v, pos_v, fbuf_v, obuf_v, gsem, ssem):
        wid = lax.axis_index("s") * _NC + lax.axis_index("c")
        pltpu.sync_copy(pos_hbm, pos_v)
        base = wid * _IPW
        pltpu.sync_copy(text_hbm.at[pl.ds(base, _IPW)], raw_v)

        def halve(m, _):
            sl = pl.ds(m * _LANES, _LANES)
            idx_v[sl] = lax.shift_right_logical(raw_v[sl], 1)
            return ()

        lax.fori_loop(0, _IPW // _LANES, halve, (), unroll=8)
        chunk0 = wid * _CPW

        def gather_start(j, b):
            ioff = pl.multiple_of(j * _CHUNK, _CHUNK)
            pltpu.async_copy(table_hbm.at[idx_v.at[pl.ds(ioff, _CHUNK)]],
                             fbuf_v.at[b], gsem.at[b])

        def gather_wait(b):
            pltpu.make_async_copy(table_hbm.at[idx_v.at[pl.ds(0, _CHUNK)]],
                                  fbuf_v.at[b], gsem.at[b]).wait()

        def store_start(j, b):
            off = pl.multiple_of((chunk0 + j) * _CHUNK, _CHUNK)
            pltpu.async_copy(obuf_v.at[b], out_hbm.at[pl.ds(off, _CHUNK)],
                             ssem.at[b])

        def store_wait(b):
            pltpu.make_async_copy(obuf_v.at[b], out_hbm.at[pl.ds(0, _CHUNK)],
                                  ssem.at[b]).wait()

        def step(j, b, first_block, last_block):
            gather_wait(b)
            if not first_block:
                store_wait(b)
            ioff = pl.multiple_of(j * _CHUNK, _CHUNK)
            p0 = lax.rem(j * _CHUNK, _C)

            def row_body(r, p):
                rsplat = jnp.full((_LANES,), ioff + r, jnp.int32)
                msk = (plsc.load_gather(raw_v, [rsplat]) & 1) == 1
                for kk in range(_D // _LANES):
                    sl = pl.ds(kk * _LANES, _LANES)
                    lo = fbuf_v[b, r, sl]
                    hi = fbuf_v[b, r, pl.ds(_D + kk * _LANES, _LANES)]
                    obuf_v[b, r, sl] = jnp.where(msk, hi, lo) + pos_v[p, sl]
                p = p + 1
                return lax.select(p == _C, 0, p)

            lax.fori_loop(0, _CHUNK, row_body, p0, unroll=2)
            if not last_block:
                gather_start(j + _NBUF, b)
            store_start(j, b)

        for b in range(_NBUF):
            gather_start(b, b)
        for b in range(_NBUF):
            step(b, b, True, False)

        def mid_block(jo, _):
            for b in range(_NBUF):
                step(jo * _NBUF + b, b, False, False)
            return ()

        lax.fori_loop(1, _BLKS - 1, mid_block, ())
        for b in range(_NBUF):
            step((_BLKS - 1) * _NBUF + b, b, False, True)
        for b in range(_NBUF):
            store_wait(b)

    return k(text_flat, table2, pos)


_TB = 2
_TPW = 244
_TBLK = _TPW // _TB
_TAILT = 7812


def _sc_build_table(tt, tail):
    mesh = plsc.VectorSubcoreMesh(core_axis_name="c", subcore_axis_name="s",
                                  num_cores=_NC, num_subcores=_NS)

    @functools.partial(
        pl.kernel,
        out_type=jax.ShapeDtypeStruct((_VOCAB // 2, _FD), jnp.float32),
        mesh=mesh,
        scratch_types=[
            pltpu.VMEM((2, _D, _TB * 128), jnp.float32),
            pltpu.VMEM((2, _TB * 64, _FD), jnp.float32),
            pltpu.VMEM((_D, _D), jnp.float32),
            pltpu.SemaphoreType.DMA((2,)),
            pltpu.SemaphoreType.DMA((2,)),
        ],
        compiler_params=pltpu.CompilerParams(use_tc_tiling_on_sc=True,
                                             needs_layout_passes=False),
    )
    def k(tt_hbm, tail_hbm, out_hbm, inb_v, ob_v, tail_v, gsem, ssem):
        wid = lax.axis_index("s") * _NC + lax.axis_index("c")
        t0 = wid * _TPW
        lanes = lax.broadcasted_iota(jnp.int32, (_LANES,), 0)

        def in_start(j, b):
            goff = pl.multiple_of((t0 + j * _TB) * 128, 128)
            pltpu.async_copy(tt_hbm.at[:, pl.ds(goff, _TB * 128)],
                             inb_v.at[b], gsem.at[b])

        def in_wait(b):
            pltpu.make_async_copy(tt_hbm.at[:, pl.ds(0, _TB * 128)],
                                  inb_v.at[b], gsem.at[b]).wait()

        def out_start(j, b):
            off = pl.multiple_of((t0 + j * _TB) * 64, 64)
            pltpu.async_copy(ob_v.at[b], out_hbm.at[pl.ds(off, _TB * 64)],
                             ssem.at[b])

        def out_wait(b):
            pltpu.make_async_copy(ob_v.at[b], out_hbm.at[pl.ds(0, _TB * 64)],
                                  ssem.at[b]).wait()

        def transpose_batch(b):
            @plsc.parallel_loop(0, _TB * 64, unroll=4)
            def _(kf):
                for h in range(2):
                    tok = jnp.full((_LANES,), 2 * kf + h, jnp.int32)
                    for kk in range(_D // _LANES):
                        dv = lanes + (kk * _LANES)
                        ob_v[b, kf, pl.ds(h * _D + kk * _LANES, _LANES)] = (
                            plsc.load_gather(inb_v.at[b], [dv, tok]))

        def step(j, b, first_block, last_block):
            in_wait(b)
            if not first_block:
                out_wait(b)
            transpose_batch(b)
            if not last_block:
                in_start(j + 2, b)
            out_start(j, b)

        for b in range(2):
            in_start(b, b)
        for b in range(2):
            step(b, b, True, False)

        def mid_block(jo, _):
            for b in range(2):
                step(jo * 2 + b, b, False, False)
            return ()

        lax.fori_loop(1, _TBLK - 1, mid_block, ())
        for b in range(2):
            step((_TBLK - 1) * 2 + b, b, False, True)
        for b in range(2):
            out_wait(b)

        @pl.when(wid == _NW - 1)
        def _():
            for e in range(2):
                goff = (_NW * _TPW + e * _TB) * 128
                pltpu.sync_copy(tt_hbm.at[:, pl.ds(goff, _TB * 128)],
                                inb_v.at[0])
                transpose_batch(0)
                pltpu.sync_copy(ob_v.at[0],
                                out_hbm.at[pl.ds((_NW * _TPW + e * _TB) * 64,
                                                 _TB * 64)])
            pltpu.sync_copy(tail_hbm, tail_v)

            def tail_row(kf, _):
                for h in range(2):
                    for kk in range(_D // _LANES):
                        sl = pl.ds(kk * _LANES, _LANES)
                        ob_v[0, kf, pl.ds(h * _D + kk * _LANES, _LANES)] = (
                            tail_v[2 * kf + h, sl])
                return ()

            lax.fori_loop(0, 32, tail_row, ())
            pltpu.sync_copy(ob_v.at[0, pl.ds(0, 32)],
                            out_hbm.at[pl.ds(_TAILT * 64, 32)])

    return k(tt, tail)


def _mask_body(o_ref):
    i = lax.broadcasted_iota(jnp.int32, (_C, _C), 0)
    j = lax.broadcasted_iota(jnp.int32, (_C, _C), 1)
    o_ref[...] = jnp.where(j > i, -jnp.inf, 0.0).astype(jnp.float32)


def _causal_mask():
    return pl.pallas_call(
        _mask_body,
        out_shape=jax.ShapeDtypeStruct((_C, _C), jnp.float32),
    )()


def kernel(text, token_embedding, positional_embedding):
    text_flat = text.astype(jnp.int32).reshape(_FLAT)
    te = token_embedding.astype(jnp.float32)
    tail = lax.slice(te, (_TAILT * 128, 0), (_VOCAB, _D))
    table2 = _sc_build_table(te.T, tail)
    x = _sc_gather_add(text_flat, table2,
                       positional_embedding.astype(jnp.float32))
    return (x.reshape(_B, _C, _D), _causal_mask())


# speedup vs baseline: 3.9557x; 2.2184x over previous
import functools

import jax
import jax.numpy as jnp
from jax import lax
from jax.experimental import pallas as pl
from jax.experimental.pallas import tpu as pltpu
from jax.experimental.pallas import tpu_sc as plsc

_VOCAB = 1000000
_C = 200
_D = 64
_B = 1024
_FLAT = _B * _C
_CHUNK = 80
_NC, _NS = 2, 16
_NW = _NC * _NS
_CPW = _FLAT // (_CHUNK * _NW)
_IPW = _CPW * _CHUNK
_NBUF = 4
_BLKS = _CPW // _NBUF
_LANES = 16
_FD = 2 * _D


def _sc_gather_add(text_flat, table2, pos):
    mesh = plsc.VectorSubcoreMesh(core_axis_name="c", subcore_axis_name="s",
                                  num_cores=_NC, num_subcores=_NS)

    @functools.partial(
        pl.kernel,
        out_type=jax.ShapeDtypeStruct((_FLAT, _D), jnp.float32),
        mesh=mesh,
        scratch_types=[
            pltpu.VMEM((_IPW,), jnp.int32),
            pltpu.VMEM((_IPW,), jnp.int32),
            pltpu.VMEM((_C, _D), jnp.float32),
            pltpu.VMEM((_NBUF, _CHUNK, _FD), jnp.float32),
            pltpu.VMEM((_NBUF, _CHUNK, _D), jnp.float32),
            pltpu.SemaphoreType.DMA((_NBUF,)),
            pltpu.SemaphoreType.DMA((_NBUF,)),
        ],
        compiler_params=pltpu.CompilerParams(use_tc_tiling_on_sc=True,
                                             needs_layout_passes=False),
    )
    def k(text_hbm, table_hbm, pos_hbm, out_hbm,
          raw_v, idx_v, pos_v, fbuf_v, obuf_v, gsem, ssem):
        wid = lax.axis_index("s") * _NC + lax.axis_index("c")
        pltpu.sync_copy(pos_hbm, pos_v)
        base = wid * _IPW
        pltpu.sync_copy(text_hbm.at[pl.ds(base, _IPW)], raw_v)

        def halve(m, _):
            sl = pl.ds(m * _LANES, _LANES)
            idx_v[sl] = lax.shift_right_logical(raw_v[sl], 1)
            return ()

        lax.fori_loop(0, _IPW // _LANES, halve, (), unroll=8)
        chunk0 = wid * _CPW

        def gather_start(j, b):
            ioff = pl.multiple_of(j * _CHUNK, _CHUNK)
            pltpu.async_copy(table_hbm.at[idx_v.at[pl.ds(ioff, _CHUNK)]],
                             fbuf_v.at[b], gsem.at[b])

        def gather_wait(b):
            pltpu.make_async_copy(table_hbm.at[idx_v.at[pl.ds(0, _CHUNK)]],
                                  fbuf_v.at[b], gsem.at[b]).wait()

        def store_start(j, b):
            off = pl.multiple_of((chunk0 + j) * _CHUNK, _CHUNK)
            pltpu.async_copy(obuf_v.at[b], out_hbm.at[pl.ds(off, _CHUNK)],
                             ssem.at[b])

        def store_wait(b):
            pltpu.make_async_copy(obuf_v.at[b], out_hbm.at[pl.ds(0, _CHUNK)],
                                  ssem.at[b]).wait()

        def step(j, b, first_block, last_block):
            gather_wait(b)
            if not first_block:
                store_wait(b)
            ioff = pl.multiple_of(j * _CHUNK, _CHUNK)
            p0 = lax.rem(j * _CHUNK, _C)

            def row_body(r, p):
                rsplat = jnp.full((_LANES,), ioff + r, jnp.int32)
                msk = (plsc.load_gather(raw_v, [rsplat]) & 1) == 1
                for kk in range(_D // _LANES):
                    sl = pl.ds(kk * _LANES, _LANES)
                    lo = fbuf_v[b, r, sl]
                    hi = fbuf_v[b, r, pl.ds(_D + kk * _LANES, _LANES)]
                    obuf_v[b, r, sl] = jnp.where(msk, hi, lo) + pos_v[p, sl]
                p = p + 1
                return lax.select(p == _C, 0, p)

            lax.fori_loop(0, _CHUNK, row_body, p0, unroll=4)
            if not last_block:
                gather_start(j + _NBUF, b)
            store_start(j, b)

        for b in range(_NBUF):
            gather_start(b, b)
        for b in range(_NBUF):
            step(b, b, True, False)

        def mid_block(jo, _):
            for b in range(_NBUF):
                step(jo * _NBUF + b, b, False, False)
            return ()

        lax.fori_loop(1, _BLKS - 1, mid_block, ())
        for b in range(_NBUF):
            step((_BLKS - 1) * _NBUF + b, b, False, True)
        for b in range(_NBUF):
            store_wait(b)

    return k(text_flat, table2, pos)


def _mask_body(o_ref):
    i = lax.broadcasted_iota(jnp.int32, (_C, _C), 0)
    j = lax.broadcasted_iota(jnp.int32, (_C, _C), 1)
    o_ref[...] = jnp.where(j > i, -jnp.inf, 0.0).astype(jnp.float32)


def _causal_mask():
    return pl.pallas_call(
        _mask_body,
        out_shape=jax.ShapeDtypeStruct((_C, _C), jnp.float32),
    )()


def kernel(text, token_embedding, positional_embedding):
    text_flat = text.astype(jnp.int32).reshape(_FLAT)
    table2 = token_embedding.astype(jnp.float32).reshape(_VOCAB // 2, _FD)
    x = _sc_gather_add(text_flat, table2,
                       positional_embedding.astype(jnp.float32))
    return (x.reshape(_B, _C, _D), _causal_mask())
